# Initial kernel scaffold; baseline (speedup 1.0000x reference)
import jax
import jax.numpy as jnp
from jax.experimental import pallas as pl


def kernel(x, edge_index, edge_attr, info_batch, params):
    raise NotImplementedError("write your pallas kernel here")


# speedup vs baseline: 16.2056x; 16.2056x over previous
import jax
import jax.numpy as jnp
from jax.experimental import pallas as pl

FN = 128
DE = 16
H = 128
GF = 128
NB = 64
CAND = 8


def _nbr_fast(edge_index, N):
    row, col = edge_index[0], edge_index[1]
    E = row.shape[0]
    order = jnp.argsort(col, stable=True).astype(jnp.int32)
    sorted_col = col[order]
    deg = jnp.bincount(col, length=N)
    start = jnp.cumsum(deg) - deg
    pos = start[row][:, None] + jnp.arange(CAND, dtype=jnp.int32)[None, :]
    in_deg = jnp.arange(CAND)[None, :] < deg[row][:, None]
    j = order[jnp.clip(pos, 0, E - 1)]
    valid = in_deg & (row[j] != col[:, None])
    rank = jnp.cumsum(valid.astype(jnp.int32), axis=1) - 1
    cols = []
    for s in range(3):
        sel = valid & (rank == s)
        cols.append(jnp.sum(jnp.where(sel, j + 1, 0), axis=1).astype(jnp.int32))
    return jnp.stack(cols, axis=1)


def _lin(p, x):
    y = x @ p['w'].T
    if 'b' in p:
        y = y + p['b']
    return y


def _layer(p, x, edge_index, edge_attr, nbr, message_edge, final, N):
    Hd = message_edge.shape[1]
    msg_tbl = jnp.concatenate([jnp.zeros((1, Hd), message_edge.dtype), message_edge], 0)
    idx = nbr.reshape(-1)
    m_g = msg_tbl[idx].reshape(nbr.shape[0], nbr.shape[1], -1)
    s_uv = m_g.sum(1)
    x_i = x[edge_index[1]]
    z_uv = jax.nn.sigmoid(_lin(p['Wmz'], jnp.concatenate([x_i, edge_attr, s_uv], -1)))
    r_uv = jax.nn.sigmoid(_lin(p['Wmr'], jnp.concatenate([x_i, edge_attr, message_edge], -1)))
    r_tbl = jnp.concatenate([jnp.zeros((1, Hd), r_uv.dtype), r_uv], 0)
    r_g = r_tbl[idx].reshape(nbr.shape[0], nbr.shape[1], -1)
    r_dash = (r_g * m_g).sum(1)
    m_dash = jnp.tanh(_lin(p['W'], jnp.concatenate([x_i, edge_attr], -1)) + _lin(p['U'], r_dash))
    m = (1.0 - z_uv) * s_uv + z_uv * m_dash
    if final:
        aggr = jax.ops.segment_sum(m, edge_index[1], num_segments=N)
        return jax.nn.relu(_lin(p['mlp'], jnp.concatenate([x, aggr], -1)))
    return m


def _seg_softmax(src, index, num_segments):
    mx = jax.ops.segment_max(src, index, num_segments=num_segments)
    mx = jnp.where(jnp.isfinite(mx), mx, 0.0)
    ex = jnp.exp(src - mx[index])
    sm = jax.ops.segment_sum(ex, index, num_segments=num_segments)
    return ex / (sm[index] + 1e-16)


def kernel(x, edge_index, edge_attr, info_batch, params):
    N = x.shape[0]
    nbr = _nbr_fast(edge_index, N)
    message_edge = jnp.zeros((edge_attr.shape[0], H), x.dtype)
    message_edge = _layer(params['conv0'], x, edge_index, edge_attr, nbr, message_edge, False, N)
    message_edge = _layer(params['conv1'], x, edge_index, edge_attr, nbr, message_edge, False, N)
    xg = _layer(params['final'], x, edge_index, edge_attr, nbr, message_edge, True, N)
    node_logits = _lin(params['nc2'], jax.nn.relu(_lin(params['nc1'], xg)))
    nei = edge_index[:, ::2]
    node_diff = jnp.abs(xg[nei[0]] - xg[nei[1]])
    node_add = xg[nei[0]] + xg[nei[1]]
    x_edge = jnp.concatenate([node_diff, node_add], 1)
    edge_logits = _lin(params['ec2'], jax.nn.relu(_lin(params['ec1'], x_edge)))
    row = edge_index[0]
    edge_batch = info_batch[row]
    cat_logits = jnp.concatenate([node_logits, edge_logits], 0)
    cat_batch = jnp.concatenate([info_batch, edge_batch[::2]], 0)
    logits = _seg_softmax(cat_logits, cat_batch, NB)
    return logits[:N], logits[N:]


# speedup vs baseline: 31.8517x; 1.9655x over previous
import functools

import jax
import jax.numpy as jnp
from jax import lax
from jax.experimental import pallas as pl
from jax.experimental.pallas import tpu as pltpu
from jax.experimental.pallas import tpu_sc as plsc

FN = 128
DE = 16
H = 128
GF = 128
NB = 64

NC = 2
NS = 16
NW = NC * NS
SUB = 128
CAND = 8
BR = CAND
JBITS = 17
_SC_PARAMS = pltpu.CompilerParams(
    needs_layout_passes=False, use_tc_tiling_on_sc=False)

_MESH = plsc.VectorSubcoreMesh(
    core_axis_name="c", subcore_axis_name="s", num_cores=NC, num_subcores=NS)


def _wid():
    return lax.axis_index("s") * NC + lax.axis_index("c")


def _lanes():
    return lax.iota(jnp.int32, 16)


TB = 8


def _build_bucket(Nn, Ep):
    EPT = Ep // TB
    CH = 2560
    NCH = EPT // CH
    ROWS = -(-(Nn + 1) // 8) * 8

    def body(colp, rowp, zer, cand_out, colbuf, rowbuf, cand, cnt, sem):
        del sem
        w = _wid()
        lanes = _lanes()
        lane0 = lanes == 0

        @pl.when(w < TB)
        def _():
            pltpu.sync_copy(zer, cand)
            pltpu.sync_copy(zer.at[pl.ds(0, ROWS)], cnt)

            def chunk_body(ci, carry):
                ebase = w * EPT + ci * CH
                pltpu.sync_copy(colp.at[pl.ds(ebase, CH)], colbuf)
                pltpu.sync_copy(rowp.at[pl.ds(ebase, CH)], rowbuf)

                def e_body(i, carry2):
                    iv = jnp.full((16,), i, jnp.int32)
                    cv = plsc.load_gather(colbuf, [iv])
                    rv = plsc.load_gather(rowbuf, [iv])
                    kv = plsc.load_gather(cnt, [cv])
                    ok = (kv < CAND) & lane0
                    addr = cv * BR + kv
                    jv = jnp.full((16,), ebase + 1, jnp.int32) + iv
                    packed = jv | (rv << JBITS)
                    plsc.store_scatter(cand, [addr], packed, mask=ok)
                    plsc.store_scatter(cnt, [cv], kv + 1, mask=lane0)
                    return carry2

                return lax.fori_loop(0, CH, e_body, carry)

            lax.fori_loop(0, NCH, chunk_body, 0)
            pltpu.sync_copy(cand, cand_out.at[pl.ds(w * (ROWS * BR), ROWS * BR)])

    return pl.kernel(
        body,
        out_type=jax.ShapeDtypeStruct((TB * ROWS * BR,), jnp.int32),
        mesh=_MESH,
        compiler_params=_SC_PARAMS,
        scratch_types=[
            pltpu.VMEM((CH,), jnp.int32),
            pltpu.VMEM((CH,), jnp.int32),
            pltpu.VMEM((ROWS * BR,), jnp.int32),
            pltpu.VMEM((ROWS,), jnp.int32),
            pltpu.SemaphoreType.DMA,
        ],
    )


def _build_extract(Nn, Ep):
    CHUNK = Ep // NW
    NSUB = CHUNK // SUB
    ROWS = -(-(Nn + 1) // 8) * 8

    def body(rowp, colp, cand2d, nbr_out, rowbuf, colbuf, idxbuf, candrows, outbuf, sem):
        w = _wid()
        base = w * CHUNK
        pltpu.sync_copy(rowp.at[pl.ds(base, CHUNK)], rowbuf)
        pltpu.sync_copy(colp.at[pl.ds(base, CHUNK)], colbuf)
        lanes = _lanes()

        def sub_body(si, carry):
            for vi in range(SUB // 16):
                uv = rowbuf[pl.ds(si * SUB + vi * 16, 16)]
                for t in range(TB):
                    idxbuf[pl.ds(t * SUB + vi * 16, 16)] = uv + t * ROWS
            cps = [
                pltpu.async_copy(
                    cand2d.at[idxbuf.at[pl.ds(t * SUB, SUB)]],
                    candrows.at[pl.ds(t * SUB, SUB)], sem)
                for t in range(TB)
            ]
            for cp in cps:
                cp.wait()
            for vi in range(SUB // 16):
                colv = colbuf[pl.ds(si * SUB + vi * 16, 16)]
                running = jnp.zeros((16,), jnp.int32)
                outs = [jnp.zeros((16,), jnp.int32) for _ in range(3)]
                rowsv = vi * 16 + lanes
                for t in range(TB):
                    trow = rowsv + t * SUB
                    for k in range(CAND):
                        kv = jnp.full((16,), k, jnp.int32)
                        candv = plsc.load_gather(candrows, [trow, kv])
                        valid = (candv != 0) & ((candv >> JBITS) != colv)
                        for s in range(3):
                            outs[s] = jnp.where(valid & (running == s), candv, outs[s])
                        running = running + valid.astype(jnp.int32)
                for s in range(3):
                    outbuf[pl.ds(s * CHUNK + si * SUB + vi * 16, 16)] = (
                        outs[s] & ((1 << JBITS) - 1))
            return carry

        lax.fori_loop(0, NSUB, sub_body, 0)
        for s in range(3):
            pltpu.sync_copy(
                outbuf.at[pl.ds(s * CHUNK, CHUNK)],
                nbr_out.at[pl.ds(s * Ep + base, CHUNK)])

    return pl.kernel(
        body,
        out_type=jax.ShapeDtypeStruct((3 * Ep,), jnp.int32),
        mesh=_MESH,
        compiler_params=_SC_PARAMS,
        scratch_types=[
            pltpu.VMEM((CHUNK,), jnp.int32),
            pltpu.VMEM((CHUNK,), jnp.int32),
            pltpu.VMEM((TB * SUB,), jnp.int32),
            pltpu.VMEM((TB * SUB, BR), jnp.int32),
            pltpu.VMEM((3 * CHUNK,), jnp.int32),
            pltpu.SemaphoreType.DMA,
        ],
    )


def _nbr_sc(edge_index, N, E):
    Ep = -(-E // (NW * SUB)) * (NW * SUB)
    row = edge_index[0]
    col = edge_index[1]
    pad = jnp.full((Ep - E,), N, jnp.int32)
    rowp = jnp.concatenate([row, pad])
    colp = jnp.concatenate([col, pad])
    ROWS = -(-(N + 1) // 8) * 8
    zer = jnp.zeros((ROWS * BR,), jnp.int32)
    cand = _build_bucket(N, Ep)(colp, rowp, zer)
    nbr3 = _build_extract(N, Ep)(rowp, colp, cand.reshape(TB * ROWS, BR)).reshape(3, Ep)
    return jnp.stack([nbr3[0, :E], nbr3[1, :E], nbr3[2, :E]], 1)


def _lin(p, x):
    y = x @ p['w'].T
    if 'b' in p:
        y = y + p['b']
    return y


def _layer(p, x, edge_index, edge_attr, nbr, message_edge, final, N):
    Hd = message_edge.shape[1]
    msg_tbl = jnp.concatenate([jnp.zeros((1, Hd), message_edge.dtype), message_edge], 0)
    idx = nbr.reshape(-1)
    m_g = msg_tbl[idx].reshape(nbr.shape[0], nbr.shape[1], -1)
    s_uv = m_g.sum(1)
    x_i = x[edge_index[1]]
    z_uv = jax.nn.sigmoid(_lin(p['Wmz'], jnp.concatenate([x_i, edge_attr, s_uv], -1)))
    r_uv = jax.nn.sigmoid(_lin(p['Wmr'], jnp.concatenate([x_i, edge_attr, message_edge], -1)))
    r_tbl = jnp.concatenate([jnp.zeros((1, Hd), r_uv.dtype), r_uv], 0)
    r_g = r_tbl[idx].reshape(nbr.shape[0], nbr.shape[1], -1)
    r_dash = (r_g * m_g).sum(1)
    m_dash = jnp.tanh(_lin(p['W'], jnp.concatenate([x_i, edge_attr], -1)) + _lin(p['U'], r_dash))
    m = (1.0 - z_uv) * s_uv + z_uv * m_dash
    if final:
        aggr = jax.ops.segment_sum(m, edge_index[1], num_segments=N)
        return jax.nn.relu(_lin(p['mlp'], jnp.concatenate([x, aggr], -1)))
    return m


def _seg_softmax(src, index, num_segments):
    mx = jax.ops.segment_max(src, index, num_segments=num_segments)
    mx = jnp.where(jnp.isfinite(mx), mx, 0.0)
    ex = jnp.exp(src - mx[index])
    sm = jax.ops.segment_sum(ex, index, num_segments=num_segments)
    return ex / (sm[index] + 1e-16)


def kernel(x, edge_index, edge_attr, info_batch, params):
    N = x.shape[0]
    E = edge_index.shape[1]
    nbr = _nbr_sc(edge_index, N, E)
    message_edge = jnp.zeros((E, H), x.dtype)
    message_edge = _layer(params['conv0'], x, edge_index, edge_attr, nbr, message_edge, False, N)
    message_edge = _layer(params['conv1'], x, edge_index, edge_attr, nbr, message_edge, False, N)
    xg = _layer(params['final'], x, edge_index, edge_attr, nbr, message_edge, True, N)
    node_logits = _lin(params['nc2'], jax.nn.relu(_lin(params['nc1'], xg)))
    nei = edge_index[:, ::2]
    node_diff = jnp.abs(xg[nei[0]] - xg[nei[1]])
    node_add = xg[nei[0]] + xg[nei[1]]
    x_edge = jnp.concatenate([node_diff, node_add], 1)
    edge_logits = _lin(params['ec2'], jax.nn.relu(_lin(params['ec1'], x_edge)))
    row = edge_index[0]
    edge_batch = info_batch[row]
    cat_logits = jnp.concatenate([node_logits, edge_logits], 0)
    cat_batch = jnp.concatenate([info_batch, edge_batch[::2]], 0)
    logits = _seg_softmax(cat_logits, cat_batch, NB)
    return logits[:N], logits[N:]
